# baseline (speedup 1.0000x reference)
import jax
import jax.numpy as jnp
from jax.experimental import pallas as pl
from jax.experimental.pallas import tpu as pltpu

_N = 10000
_D_IN = 128
_D_HID = 64
_D_OUT = 16

_BP = 1000
_BI = 400
_GP = _N // _BP
_GI = _N // _BI


def _proj_kernel(x_ref, w1_ref, b1_ref, wg_ref, p_ref):
    h = jnp.dot(x_ref[...], w1_ref[...], preferred_element_type=jnp.float32)
    h = jnp.maximum(h + b1_ref[...], 0.0)
    p_ref[...] = jnp.dot(h, wg_ref[...], preferred_element_type=jnp.float32)


def _agg_kernel(conn_ref, p_ref, bg_ref, out_ref):
    out_ref[...] = jnp.dot(conn_ref[...], p_ref[...],
                           preferred_element_type=jnp.float32) + bg_ref[...]


def kernel(x, conn, W1, b1, Wg, bg):
    p = pl.pallas_call(
        _proj_kernel,
        grid=(_GP,),
        in_specs=[
            pl.BlockSpec((_BP, _D_IN), lambda i: (i, 0)),
            pl.BlockSpec((_D_IN, _D_HID), lambda i: (0, 0)),
            pl.BlockSpec((1, _D_HID), lambda i: (0, 0)),
            pl.BlockSpec((_D_HID, _D_OUT), lambda i: (0, 0)),
        ],
        out_specs=pl.BlockSpec((_BP, _D_OUT), lambda i: (i, 0)),
        out_shape=jax.ShapeDtypeStruct((_N, _D_OUT), jnp.float32),
    )(x, W1, b1.reshape(1, _D_HID), Wg)

    out = pl.pallas_call(
        _agg_kernel,
        grid=(_GI,),
        in_specs=[
            pl.BlockSpec((_BI, _N), lambda i: (i, 0)),
            pl.BlockSpec((_N, _D_OUT), lambda i: (0, 0)),
            pl.BlockSpec((1, _D_OUT), lambda i: (0, 0)),
        ],
        out_specs=pl.BlockSpec((_BI, _D_OUT), lambda i: (i, 0)),
        out_shape=jax.ShapeDtypeStruct((_N, _D_OUT), jnp.float32),
        compiler_params=pltpu.CompilerParams(
            dimension_semantics=("arbitrary",)),
    )(conn, p, bg.reshape(1, _D_OUT))
    return out


# speedup vs baseline: 1.0757x; 1.0757x over previous
import jax
import jax.numpy as jnp
from jax.experimental import pallas as pl
from jax.experimental.pallas import tpu as pltpu

_N = 10000
_D_IN = 128
_D_HID = 64
_D_OUT = 16

_BI = 400
_GI = _N // _BI


def _fused_kernel(x_ref, conn_ref, w1_ref, b1_ref, wg_ref, bg_ref,
                  out_ref, p_ref):
    @pl.when(pl.program_id(0) == 0)
    def _proj():
        h = jnp.dot(x_ref[...], w1_ref[...],
                    preferred_element_type=jnp.float32)
        h = jnp.maximum(h + b1_ref[...], 0.0)
        p_ref[...] = jnp.dot(h, wg_ref[...],
                             preferred_element_type=jnp.float32)

    out_ref[...] = jnp.dot(conn_ref[...], p_ref[...],
                           preferred_element_type=jnp.float32) + bg_ref[...]


def kernel(x, conn, W1, b1, Wg, bg):
    return pl.pallas_call(
        _fused_kernel,
        grid=(_GI,),
        in_specs=[
            pl.BlockSpec((_N, _D_IN), lambda i: (0, 0)),
            pl.BlockSpec((_BI, _N), lambda i: (i, 0)),
            pl.BlockSpec((_D_IN, _D_HID), lambda i: (0, 0)),
            pl.BlockSpec((1, _D_HID), lambda i: (0, 0)),
            pl.BlockSpec((_D_HID, _D_OUT), lambda i: (0, 0)),
            pl.BlockSpec((1, _D_OUT), lambda i: (0, 0)),
        ],
        out_specs=pl.BlockSpec((_BI, _D_OUT), lambda i: (i, 0)),
        out_shape=jax.ShapeDtypeStruct((_N, _D_OUT), jnp.float32),
        scratch_shapes=[pltpu.VMEM((_N, _D_OUT), jnp.float32)],
        compiler_params=pltpu.CompilerParams(
            dimension_semantics=("arbitrary",)),
    )(x, conn, W1, b1.reshape(1, _D_HID), Wg, bg.reshape(1, _D_OUT))
